# Initial kernel scaffold; baseline (speedup 1.0000x reference)
import jax
import jax.numpy as jnp
from jax import lax
from jax.experimental import pallas as pl
from jax.experimental.pallas import tpu as pltpu
from jax.experimental.pallas import tpu_sc as plsc

N_NODES = 10000
N_EDGES = 320000
D = 128

NC = 2
NS = 16
CHUNK = 128
NCHUNK = 80
EDGES_PER_TILE = CHUNK * NCHUNK
PADDED_E = NC * NS * EDGES_PER_TILE
N_SP = 10240
ROWS_PER_TILE = N_SP // NS
TRASH_ROW = N_NODES


def _sc_agg_kernel_body(table_hbm, src_hbm, dst_hbm, out_hbm,
                        src_v, dst_v, rows_a, rows_b, agg_sp, sem_a, sem_b):
    c = lax.axis_index("c")
    s = lax.axis_index("s")

    pltpu.sync_copy(src_hbm.at[c, s], src_v)
    pltpu.sync_copy(dst_hbm.at[c, s], dst_v)

    def _zero_row(i, carry):
        for j in range(D // 16):
            rows_a[i, pl.ds(j * 16, 16)] = jnp.zeros((16,), jnp.float32)
        return carry

    lax.fori_loop(0, CHUNK, _zero_row, 0)
    for k in range(ROWS_PER_TILE // CHUNK):
        pltpu.sync_copy(rows_a, agg_sp.at[pl.ds(s * ROWS_PER_TILE + k * CHUNK, CHUNK)])

    plsc.subcore_barrier()

    pltpu.async_copy(table_hbm.at[src_v.at[0]], rows_a, sem_a)

    def _body(jj, carry):
        j0 = jj * 2
        pltpu.async_copy(table_hbm.at[src_v.at[j0 + 1]], rows_b, sem_b)
        pltpu.make_async_copy(table_hbm.at[src_v.at[j0]], rows_a, sem_a).wait()
        pltpu.sync_copy(rows_a, agg_sp.at[dst_v.at[j0]], add=True)

        @pl.when(j0 + 2 < NCHUNK)
        def _():
            pltpu.async_copy(table_hbm.at[src_v.at[j0 + 2]], rows_a, sem_a)

        pltpu.make_async_copy(table_hbm.at[src_v.at[j0 + 1]], rows_b, sem_b).wait()
        pltpu.sync_copy(rows_b, agg_sp.at[dst_v.at[j0 + 1]], add=True)
        return carry

    lax.fori_loop(0, NCHUNK // 2, _body, 0)

    plsc.subcore_barrier()
    pltpu.sync_copy(agg_sp.at[pl.ds(s * ROWS_PER_TILE, ROWS_PER_TILE)],
                    out_hbm.at[c, pl.ds(s * ROWS_PER_TILE, ROWS_PER_TILE)])


def _sc_aggregate(table, src_p, dst_p):
    mesh = plsc.VectorSubcoreMesh(core_axis_name="c", subcore_axis_name="s")
    fn = pl.kernel(
        _sc_agg_kernel_body,
        out_type=jax.ShapeDtypeStruct((NC, N_SP, D), jnp.float32),
        mesh=mesh,
        scratch_types=[
            pltpu.VMEM((NCHUNK, CHUNK), jnp.int32),
            pltpu.VMEM((NCHUNK, CHUNK), jnp.int32),
            pltpu.VMEM((CHUNK, D), jnp.float32),
            pltpu.VMEM((CHUNK, D), jnp.float32),
            pltpu.VMEM_SHARED((N_SP, D), jnp.float32),
            pltpu.SemaphoreType.DMA,
            pltpu.SemaphoreType.DMA,
        ],
    )
    return fn(table, src_p, dst_p)


def _tc_stage1_body(x_ref, agg_ref, w_ref, b_ref, g_ref, beta_ref, out_ref):
    agg = agg_ref[0, :N_NODES, :] + agg_ref[1, :N_NODES, :]
    h = x_ref[...] + agg
    h = lax.dot_general(h, w_ref[...], (((1,), (1,)), ((), ())),
                        preferred_element_type=jnp.float32)
    h = h + b_ref[...]
    mu = jnp.mean(h, axis=0, keepdims=True)
    var = jnp.mean((h - mu) ** 2, axis=0, keepdims=True)
    hn = (h - mu) * lax.rsqrt(var + 1e-5) * g_ref[...] + beta_ref[...]
    out_ref[...] = jnp.maximum(hn, 0.0)


def _tc_stage2_body(x_ref, agg_ref, w_ref, b_ref, g_ref, beta_ref, out_ref):
    agg = agg_ref[0, :N_NODES, :] + agg_ref[1, :N_NODES, :]
    h = x_ref[...] + agg
    h = lax.dot_general(h, w_ref[...], (((1,), (1,)), ((), ())),
                        preferred_element_type=jnp.float32)
    h = h + b_ref[...]
    mu = jnp.mean(h, axis=0, keepdims=True)
    var = jnp.mean((h - mu) ** 2, axis=0, keepdims=True)
    hn = (h - mu) * lax.rsqrt(var + 1e-5) * g_ref[...] + beta_ref[...]
    m = jnp.max(hn, axis=1, keepdims=True)
    lse = jnp.log(jnp.sum(jnp.exp(hn - m), axis=1, keepdims=True)) + m
    out_ref[...] = hn - lse


def _tc_stage(body, x, agg, w, b, g, beta):
    return pl.pallas_call(
        body,
        out_shape=jax.ShapeDtypeStruct((N_NODES, D), jnp.float32),
    )(x, agg, w, b.reshape(1, D), g.reshape(1, D), beta.reshape(1, D))


def kernel(x, edge_index, W1, b1, W2, b2,
           bn1_gamma, bn1_beta, bn2_gamma, bn2_beta):
    src = edge_index[0].astype(jnp.int32)
    dst = edge_index[1].astype(jnp.int32)
    pad = PADDED_E - N_EDGES
    src_p = jnp.concatenate([src, jnp.zeros((pad,), jnp.int32)]
                            ).reshape(NC, NS, NCHUNK, CHUNK)
    dst_p = jnp.concatenate([dst, jnp.full((pad,), TRASH_ROW, jnp.int32)]
                            ).reshape(NC, NS, NCHUNK, CHUNK)

    agg1 = _sc_aggregate(x, src_p, dst_p)
    h1 = _tc_stage(_tc_stage1_body, x, agg1, W1, b1, bn1_gamma, bn1_beta)
    agg2 = _sc_aggregate(h1, src_p, dst_p)
    out = _tc_stage(_tc_stage2_body, h1, agg2, W2, b2, bn2_gamma, bn2_beta)
    return out


# speedup vs baseline: 3.3060x; 3.3060x over previous
import jax
import jax.numpy as jnp
from jax import lax
from jax.experimental import pallas as pl
from jax.experimental.pallas import tpu as pltpu
from jax.experimental.pallas import tpu_sc as plsc

N_NODES = 10000
N_EDGES = 320000
D = 128

NC = 2
NS = 16
CHUNK = 128
NCHUNK = 80
IDX_BLK = 16
NSLAB = NCHUNK // IDX_BLK
EDGES_PER_TILE = CHUNK * NCHUNK
PADDED_E = NC * NS * EDGES_PER_TILE
N_SP = 10240
ROWS_PER_TILE = N_SP // NS
TRASH_ROW = N_NODES


def _sc_agg_kernel_body(table_hbm, src_hbm, dst_hbm, out_hbm,
                        src_v, dst_v, rows_a, rows_b, agg_sp, sem_a, sem_b):
    c = lax.axis_index("c")
    s = lax.axis_index("s")

    def _zero_row(i, carry):
        for j in range(D // 16):
            rows_a[i, pl.ds(j * 16, 16)] = jnp.zeros((16,), jnp.float32)
        return carry

    lax.fori_loop(0, CHUNK, _zero_row, 0)
    for k in range(ROWS_PER_TILE // CHUNK):
        pltpu.sync_copy(rows_a, agg_sp.at[pl.ds(s * ROWS_PER_TILE + k * CHUNK, CHUNK)])

    plsc.subcore_barrier()

    def _slab(sl, carry):
        pltpu.sync_copy(src_hbm.at[c, s, sl], src_v)
        pltpu.sync_copy(dst_hbm.at[c, s, sl], dst_v)
        pltpu.async_copy(table_hbm.at[src_v.at[0]], rows_a, sem_a)

        def _body(jj, carry2):
            j0 = jj * 2
            pltpu.async_copy(table_hbm.at[src_v.at[j0 + 1]], rows_b, sem_b)
            pltpu.make_async_copy(table_hbm.at[src_v.at[j0]], rows_a, sem_a).wait()
            pltpu.sync_copy(rows_a, agg_sp.at[dst_v.at[j0]], add=True)

            @pl.when(j0 + 2 < IDX_BLK)
            def _():
                pltpu.async_copy(table_hbm.at[src_v.at[j0 + 2]], rows_a, sem_a)

            pltpu.make_async_copy(table_hbm.at[src_v.at[j0 + 1]], rows_b, sem_b).wait()
            pltpu.sync_copy(rows_b, agg_sp.at[dst_v.at[j0 + 1]], add=True)
            return carry2

        lax.fori_loop(0, IDX_BLK // 2, _body, 0)
        return carry

    lax.fori_loop(0, NSLAB, _slab, 0)

    plsc.subcore_barrier()
    pltpu.sync_copy(agg_sp.at[pl.ds(s * ROWS_PER_TILE, ROWS_PER_TILE)],
                    out_hbm.at[c, pl.ds(s * ROWS_PER_TILE, ROWS_PER_TILE)])


def _sc_aggregate(table, src_p, dst_p):
    mesh = plsc.VectorSubcoreMesh(core_axis_name="c", subcore_axis_name="s")
    fn = pl.kernel(
        _sc_agg_kernel_body,
        out_type=jax.ShapeDtypeStruct((NC, N_SP, D), jnp.float32),
        mesh=mesh,
        scratch_types=[
            pltpu.VMEM((IDX_BLK, CHUNK), jnp.int32),
            pltpu.VMEM((IDX_BLK, CHUNK), jnp.int32),
            pltpu.VMEM((CHUNK, D), jnp.float32),
            pltpu.VMEM((CHUNK, D), jnp.float32),
            pltpu.VMEM_SHARED((N_SP, D), jnp.float32),
            pltpu.SemaphoreType.DMA,
            pltpu.SemaphoreType.DMA,
        ],
    )
    return fn(table, src_p, dst_p)


def _tc_stage1_body(x_ref, agg_ref, w_ref, b_ref, g_ref, beta_ref, out_ref):
    agg = agg_ref[0, :N_NODES, :] + agg_ref[1, :N_NODES, :]
    h = x_ref[...] + agg
    h = lax.dot_general(h, w_ref[...], (((1,), (1,)), ((), ())),
                        preferred_element_type=jnp.float32)
    h = h + b_ref[...]
    mu = jnp.mean(h, axis=0, keepdims=True)
    var = jnp.mean((h - mu) ** 2, axis=0, keepdims=True)
    hn = (h - mu) * lax.rsqrt(var + 1e-5) * g_ref[...] + beta_ref[...]
    out_ref[...] = jnp.maximum(hn, 0.0)


def _tc_stage2_body(x_ref, agg_ref, w_ref, b_ref, g_ref, beta_ref, out_ref):
    agg = agg_ref[0, :N_NODES, :] + agg_ref[1, :N_NODES, :]
    h = x_ref[...] + agg
    h = lax.dot_general(h, w_ref[...], (((1,), (1,)), ((), ())),
                        preferred_element_type=jnp.float32)
    h = h + b_ref[...]
    mu = jnp.mean(h, axis=0, keepdims=True)
    var = jnp.mean((h - mu) ** 2, axis=0, keepdims=True)
    hn = (h - mu) * lax.rsqrt(var + 1e-5) * g_ref[...] + beta_ref[...]
    m = jnp.max(hn, axis=1, keepdims=True)
    lse = jnp.log(jnp.sum(jnp.exp(hn - m), axis=1, keepdims=True)) + m
    out_ref[...] = hn - lse


def _tc_stage(body, x, agg, w, b, g, beta):
    return pl.pallas_call(
        body,
        out_shape=jax.ShapeDtypeStruct((N_NODES, D), jnp.float32),
    )(x, agg, w, b.reshape(1, D), g.reshape(1, D), beta.reshape(1, D))


def kernel(x, edge_index, W1, b1, W2, b2,
           bn1_gamma, bn1_beta, bn2_gamma, bn2_beta):
    src = edge_index[0].astype(jnp.int32)
    dst = edge_index[1].astype(jnp.int32)
    pad = PADDED_E - N_EDGES
    src_p = jnp.concatenate([src, jnp.zeros((pad,), jnp.int32)]
                            ).reshape(NC, NS, NSLAB, IDX_BLK, CHUNK)
    dst_p = jnp.concatenate([dst, jnp.full((pad,), TRASH_ROW, jnp.int32)]
                            ).reshape(NC, NS, NSLAB, IDX_BLK, CHUNK)

    agg1 = _sc_aggregate(x, src_p, dst_p)
    h1 = _tc_stage(_tc_stage1_body, x, agg1, W1, b1, bn1_gamma, bn1_beta)
    agg2 = _sc_aggregate(h1, src_p, dst_p)
    out = _tc_stage(_tc_stage2_body, h1, agg2, W2, b2, bn2_gamma, bn2_beta)
    return out
